# Initial kernel scaffold; baseline (speedup 1.0000x reference)
import jax
import jax.numpy as jnp
from jax.experimental import pallas as pl


def kernel(x, pos, batch, reflectance, sf, params):
    raise NotImplementedError("write your pallas kernel here")


# speedup vs baseline: 4.6697x; 4.6697x over previous
import functools

import jax
import jax.numpy as jnp
from jax import lax
from jax.experimental import pallas as pl
from jax.experimental.pallas import tpu as pltpu
from jax.experimental.pallas import tpu_sc as plsc

K = 16
TW = 256
_BIG = 3.0e38
_SQ3 = 0.5773502691896258

_NC, _NS = 2, 16
_NW = _NC * _NS
_CHUNK = 128


def _table_body(x_ref, p4_ref, w_ref, t_ref):
    t_ref[:, 0:128] = jnp.dot(x_ref[...], w_ref[...],
                              preferred_element_type=jnp.float32)
    t_ref[:, 128:132] = p4_ref[...]
    t_ref[:, 132:TW] = jnp.zeros((x_ref.shape[0], TW - 132), jnp.float32)


def _build_table(x, pos4, wx, blk=1000):
    n, d = x.shape
    return pl.pallas_call(
        _table_body,
        grid=(n // blk,),
        in_specs=[
            pl.BlockSpec((blk, d), lambda i: (i, 0)),
            pl.BlockSpec((blk, 4), lambda i: (i, 0)),
            pl.BlockSpec((d, d), lambda i: (0, 0)),
        ],
        out_specs=pl.BlockSpec((blk, TW), lambda i: (i, 0)),
        out_shape=jax.ShapeDtypeStruct((n, TW), jnp.float32),
    )(x, pos4, wx)


def _bf16_rtne(v):
    u = lax.bitcast_convert_type(v, jnp.uint32)
    r = u + jnp.uint32(0x7FFF) + ((u >> 16) & jnp.uint32(1))
    return lax.bitcast_convert_type(r & jnp.uint32(0xFFFF0000), jnp.float32)


def _knn_body(y_ref, pt_ref, nbr_ref, d2_ref):
    cb = y_ref.shape[0]
    n = pt_ref.shape[1]
    y0, y1, y2 = y_ref[:, 0:1], y_ref[:, 1:2], y_ref[:, 2:3]
    p0, p1, p2 = pt_ref[0:1, :], pt_ref[1:2, :], pt_ref[2:3, :]
    ysq = (y0 * y0 + y1 * y1) + y2 * y2
    psq = (p0 * p0 + p1 * p1) + p2 * p2
    yb0, yb1, yb2 = _bf16_rtne(y0), _bf16_rtne(y1), _bf16_rtne(y2)
    pb0, pb1, pb2 = _bf16_rtne(p0), _bf16_rtne(p1), _bf16_rtne(p2)
    cross = (yb0 * pb0 + yb1 * pb1) + yb2 * pb2
    d2_ref[...] = (ysq + psq) - 2.0 * cross
    iota = lax.broadcasted_iota(jnp.int32, (cb, n), 1)
    cols = []
    for _ in range(K):
        cur = d2_ref[...]
        am = jnp.argmin(cur, axis=1).astype(jnp.int32)
        cols.append(am[:, None])
        d2_ref[...] = jnp.where(iota == am[:, None], _BIG, cur)
    nbr_ref[...] = jnp.concatenate(cols, axis=1)


def _knn(y_pad, pos_t, blk=128):
    mp = y_pad.shape[0]
    n = pos_t.shape[1]
    return pl.pallas_call(
        _knn_body,
        grid=(mp // blk,),
        in_specs=[
            pl.BlockSpec((blk, 3), lambda i: (i, 0)),
            pl.BlockSpec((8, n), lambda i: (0, 0)),
        ],
        out_specs=pl.BlockSpec((blk, K), lambda i: (i, 0)),
        out_shape=jax.ShapeDtypeStruct((mp, K), jnp.int32),
        scratch_shapes=[pltpu.VMEM((blk, n), jnp.float32)],
    )(y_pad, pos_t)


def _sc_gather_call(table, col):
    edges = col.shape[0]
    per_w = edges // _NW
    n_chunks = per_w // _CHUNK
    mesh = plsc.VectorSubcoreMesh(core_axis_name="c", subcore_axis_name="s")

    @functools.partial(
        pl.kernel,
        mesh=mesh,
        out_type=jax.ShapeDtypeStruct((edges, TW), jnp.float32),
        scratch_types=[
            pltpu.VMEM((_CHUNK,), jnp.int32),
            pltpu.VMEM((_CHUNK, TW), jnp.float32),
            pltpu.SemaphoreType.DMA,
        ],
    )
    def gather_k(table_hbm, col_hbm, out_hbm, idx_v, rows_v, sem):
        wid = lax.axis_index("s") * _NC + lax.axis_index("c")
        base0 = wid * per_w

        def body(i, carry):
            base = base0 + i * _CHUNK
            pltpu.sync_copy(col_hbm.at[pl.ds(base, _CHUNK)], idx_v)
            pltpu.async_copy(table_hbm.at[idx_v], rows_v, sem).wait()
            pltpu.sync_copy(rows_v, out_hbm.at[pl.ds(base, _CHUNK)])
            return carry

        lax.fori_loop(0, n_chunks, body, 0)

    return gather_k(table, col)


def _block_body(g_ref, pc_ref, wrs_ref, lkp_ref, lkb_ref, vecs_ref,
                a1_ref, a2_ref, a3_ref, a4_ref, out_ref):
    cb = pc_ref.shape[0]
    cbk = cb * K
    d = 128

    xw = g_ref[:, 0:128]
    pcol = g_ref[:, 128:132]
    pc = pc_ref[...]
    pcr = jnp.broadcast_to(pc[:, None, :], (cb, K, 4)).reshape(cbk, 4)
    rel = pcol - pcr

    r0, r1, r2 = rel[:, 0:1], rel[:, 1:2], rel[:, 2:3]
    nrm = jnp.sqrt(r0 * r0 + r1 * r1 + r2 * r2)
    inv = _SQ3 / (nrm + 1e-8)
    d0, d1, d2 = r0 * inv, r1 * inv, r2 * inv
    resp = jnp.concatenate([
        d0 + d1 + d2, -d0 + d1 + d2, d0 - d1 + d2, d0 + d1 - d2,
        -d0 - d1 + d2, -d0 + d1 - d2, d0 - d1 - d2, -d0 - d1 - d2,
    ], axis=1)
    nshape = jnp.mean(resp.reshape(cb, K, 8), axis=1)

    q = jnp.broadcast_to(lkb_ref[0:1, 0:3], (cb, 3))
    for j in range(8):
        q = q + nshape[:, j:j + 1] * lkp_ref[j:j + 1, 0:3]
    mu = jnp.mean(q, axis=1, keepdims=True)
    var = jnp.mean((q - mu) ** 2, axis=1, keepdims=True)
    sfe = (lkb_ref[1:2, 0:3] * (q - mu) * lax.rsqrt(var + 1e-5)
           + lkb_ref[2:3, 0:3])

    smat = (sfe[:, 0:1] * wrs_ref[4:5, :] + sfe[:, 1:2] * wrs_ref[5:6, :]
            + sfe[:, 2:3] * wrs_ref[6:7, :] + wrs_ref[7:8, :])
    smat_r = jnp.broadcast_to(smat[:, None, :], (cb, K, d)).reshape(cbk, d)
    relc = (rel[:, 0:1] * wrs_ref[0:1, :] + rel[:, 1:2] * wrs_ref[1:2, :]
            + rel[:, 2:3] * wrs_ref[2:3, :] + rel[:, 3:4] * wrs_ref[3:4, :])
    hpre = xw + relc + smat_r
    h = hpre * jax.nn.sigmoid(hpre)
    agg = jnp.max(h.reshape(cb, K, d), axis=1)

    t = jnp.dot(agg, a1_ref[...], preferred_element_type=jnp.float32)
    t = t + vecs_ref[0:1, :]
    t = t * jax.nn.sigmoid(t)
    t = t * vecs_ref[1:2, :] + vecs_ref[2:3, :]
    t = t * jax.nn.sigmoid(t)
    t = jnp.dot(t, a2_ref[...], preferred_element_type=jnp.float32)
    t = t + vecs_ref[3:4, :]
    t = t * jax.nn.sigmoid(t)
    t = t * vecs_ref[4:5, :] + vecs_ref[5:6, :]
    t = t * jax.nn.sigmoid(t)
    t = t * vecs_ref[6:7, :] + vecs_ref[7:8, :]
    t = t * jax.nn.sigmoid(t)
    t = jnp.dot(t, a3_ref[...], preferred_element_type=jnp.float32)
    t = t + vecs_ref[8:9, :]
    t = t * jax.nn.sigmoid(t)
    o = jnp.dot(t, a4_ref[...], preferred_element_type=jnp.float32)
    o = o + vecs_ref[9:10, 0:128] + agg
    out_ref[...] = o * jax.nn.sigmoid(o)


def _block(g, pc_pad, wrs, lkp, lkb, vecs, a1, a2, a3, a4, cb=256):
    mp = pc_pad.shape[0]
    exp = a2.shape[0]
    return pl.pallas_call(
        _block_body,
        grid=(mp // cb,),
        in_specs=[
            pl.BlockSpec((cb * K, TW), lambda i: (i, 0)),
            pl.BlockSpec((cb, 4), lambda i: (i, 0)),
            pl.BlockSpec((8, 128), lambda i: (0, 0)),
            pl.BlockSpec((8, 128), lambda i: (0, 0)),
            pl.BlockSpec((8, 128), lambda i: (0, 0)),
            pl.BlockSpec((16, exp), lambda i: (0, 0)),
            pl.BlockSpec((128, exp), lambda i: (0, 0)),
            pl.BlockSpec((exp, exp), lambda i: (0, 0)),
            pl.BlockSpec((exp, exp), lambda i: (0, 0)),
            pl.BlockSpec((exp, 128), lambda i: (0, 0)),
        ],
        out_specs=pl.BlockSpec((cb, 128), lambda i: (i, 0)),
        out_shape=jax.ShapeDtypeStruct((mp, 128), jnp.float32),
    )(g, pc_pad, wrs, lkp, lkb, vecs, a1, a2, a3, a4)


def kernel(x, pos, batch, reflectance, sf, params):
    p = params
    n, d = x.shape
    m = n // 2
    mp = ((m + 255) // 256) * 256

    s = jnp.sqrt(jnp.float32(1.0 + 1e-5))

    gsm = p['mlp_bn_g'] / s
    wx = p['mlp_W'][0:d] * gsm[None, :]
    wr = p['mlp_W'][d:d + 4] * gsm[None, :]
    ws = p['mlp_W'][d + 4:d + 7] * gsm[None, :]
    bf = p['mlp_b'] * gsm + p['mlp_bn_b']
    wrs = jnp.concatenate([wr, ws, bf[None, :]], axis=0)

    gse = p['exp_bn_g'] / s
    a1 = p['exp_W'] * gse[None, :]
    c1 = p['exp_b'] * gse + p['exp_bn_b']
    gs1d = p['ds1_dw_bn_g'] / s
    d1 = p['ds1_dw_w'] * gs1d
    e1 = p['ds1_dw_b'] * gs1d + p['ds1_dw_bn_b']
    gs1p = p['ds1_pw_bn_g'] / s
    a2 = p['ds1_pw_W'] * gs1p[None, :]
    c2 = p['ds1_pw_b'] * gs1p + p['ds1_pw_bn_b']
    gmid = p['mid_bn_g'] / s
    bmid = p['mid_bn_b']
    gs2d = p['ds2_dw_bn_g'] / s
    d2s = p['ds2_dw_w'] * gs2d
    e2 = p['ds2_dw_b'] * gs2d + p['ds2_dw_bn_b']
    gs2p = p['ds2_pw_bn_g'] / s
    a3 = p['ds2_pw_W'] * gs2p[None, :]
    c3 = p['ds2_pw_b'] * gs2p + p['ds2_pw_bn_b']
    g2 = p['bn2_g'] / s
    b2 = p['bn2_b']
    gp = p['proj_bn_g'] / s
    p2 = p['proj_W'] * gp[None, :]
    a4 = g2[:, None] * p2
    c4 = b2 @ p2 + p['proj_b'] * gp + p['proj_bn_b']

    exp = a2.shape[0]
    vecs = jnp.zeros((16, exp), jnp.float32)
    for i, v in enumerate([c1, d1, e1, c2, gmid, bmid, d2s, e2, c3]):
        vecs = vecs.at[i, :].set(v)
    vecs = vecs.at[9, 0:d].set(c4)

    lkp = jnp.zeros((8, 128), jnp.float32).at[:, 0:3].set(p['lk_W'])
    lkb = (jnp.zeros((8, 128), jnp.float32)
           .at[0, 0:3].set(p['lk_b'])
           .at[1, 0:3].set(p['lk_ln_g'])
           .at[2, 0:3].set(p['lk_ln_b']))

    pos3 = pos[:, :3]
    pos4 = jnp.concatenate([pos3, reflectance[:, None]], axis=-1)

    scores = (x @ p['sampler_W'])[:, 0] + p['sampler_b'][0]
    _, idx = lax.top_k(scores, m)

    yv = pos3[idx]
    y_pad = jnp.zeros((mp, 3), jnp.float32).at[0:m].set(yv)
    pc_pad = jnp.zeros((mp, 4), jnp.float32).at[0:m].set(pos4[idx])
    pos_t = jnp.zeros((8, n), jnp.float32).at[0:3].set(pos3.T)

    table = _build_table(x, pos4, wx)
    nbr = _knn(y_pad, pos_t)
    col = nbr.reshape(-1)
    g = _sc_gather_call(table, col)
    out_full = _block(g, pc_pad, wrs, lkp, lkb, vecs, a1, a2, a3, a4)

    out = out_full[0:m]
    return (out, yv, batch[idx], reflectance[idx], sf)
